# baseline (speedup 1.0000x reference)
import functools

import jax
import jax.numpy as jnp
from jax import lax
from jax.experimental import pallas as pl
from jax.experimental.pallas import tpu as pltpu
from jax.experimental.pallas import tpu_sc as plsc

K_EDGE = 1024
R_SPAN = 256
GC = 128


def _mm_body(x_ref, w_ref, a2_ref, h_ref, s_ref):
    h = jnp.dot(x_ref[...], w_ref[...], preferred_element_type=jnp.float32)
    h_ref[...] = h.astype(jnp.bfloat16)
    s_ref[...] = jnp.dot(h, a2_ref[...], preferred_element_type=jnp.float32)


def _transform(x, W, A2, row_blk):
    N, F = x.shape
    U = W.shape[1]
    grid = (N // row_blk,)
    return pl.pallas_call(
        _mm_body,
        grid=grid,
        in_specs=[
            pl.BlockSpec((row_blk, F), lambda b: (b, 0)),
            pl.BlockSpec((F, U), lambda b: (0, 0)),
            pl.BlockSpec((U, 2), lambda b: (0, 0)),
        ],
        out_specs=[
            pl.BlockSpec((row_blk, U), lambda b: (b, 0)),
            pl.BlockSpec((row_blk, 2), lambda b: (b, 0)),
        ],
        out_shape=[
            jax.ShapeDtypeStruct((N, U), jnp.bfloat16),
            jax.ShapeDtypeStruct((N, 2), jnp.float32),
        ],
    )(x, W, A2)


def _make_sc_edges(N, UW, E, E_pad):
    info = plsc.get_sparse_core_info()
    NC, NS, L = info.num_cores, info.num_subcores, info.num_lanes
    NW = NC * NS
    EC = E_pad // NW
    n_full = EC // GC
    rem = EC % GC

    mesh = plsc.VectorSubcoreMesh(core_axis_name="c", subcore_axis_name="s")

    @functools.partial(
        pl.kernel,
        mesh=mesh,
        compiler_params=pltpu.CompilerParams(needs_layout_passes=False),
        out_type=[
            jax.ShapeDtypeStruct((E_pad,), jnp.float32),
            jax.ShapeDtypeStruct((E_pad, UW), jnp.int32),
        ],
        scratch_types=[
            pltpu.VMEM((N,), jnp.float32),
            pltpu.VMEM((N,), jnp.float32),
            pltpu.VMEM((EC,), jnp.int32),
            pltpu.VMEM((EC,), jnp.int32),
            pltpu.VMEM((EC,), jnp.float32),
            pltpu.VMEM((2, GC, UW), jnp.int32),
            pltpu.SemaphoreType.DMA((2,)),
        ],
    )
    def sc_edges(s_hbm, src_hbm, dst_hbm, h_hbm, score_hbm, g_hbm,
                 s1_v, s2_v, src_v, dst_v, score_v, rows_v, sem):
        wid = lax.axis_index("s") * NC + lax.axis_index("c")
        base = wid * EC
        pltpu.sync_copy(s_hbm.at[0], s1_v)
        pltpu.sync_copy(s_hbm.at[1], s2_v)
        pltpu.sync_copy(src_hbm.at[pl.ds(base, EC)], src_v)
        pltpu.sync_copy(dst_hbm.at[pl.ds(base, EC)], dst_v)

        def score_body(i, carry):
            o = i * L
            sv = src_v[pl.ds(o, L)]
            dv = dst_v[pl.ds(o, L)]
            t = plsc.load_gather(s1_v, [sv]) + plsc.load_gather(s2_v, [dv])
            t = jnp.maximum(t, 0.2 * t)
            t = jnp.clip(t, -2.0, 2.0)
            sc = jnp.exp(t)
            gid = base + o + lax.iota(jnp.int32, L)
            sc = jnp.where(gid < E, sc, 0.0)
            score_v[pl.ds(o, L)] = sc
            return carry

        lax.fori_loop(0, EC // L, score_body, 0)
        pltpu.sync_copy(score_v, score_hbm.at[pl.ds(base, EC)])

        def _start(k, b):
            pltpu.async_copy(h_hbm.at[dst_v.at[pl.ds(k * GC, GC)]],
                             rows_v.at[b], sem.at[b])

        def _wait(k, b):
            pltpu.make_async_copy(h_hbm.at[dst_v.at[pl.ds(k * GC, GC)]],
                                  rows_v.at[b], sem.at[b]).wait()

        _start(0, 0)
        if n_full > 1:
            _start(1, 1)

        def pair_body(p, carry):
            k0 = p * 2
            for b in range(2):
                k = k0 + b
                _wait(k, b)
                pltpu.sync_copy(rows_v.at[b],
                                g_hbm.at[pl.ds(base + k * GC, GC)])
                nk = k + 2

                @pl.when(nk < n_full)
                def _():
                    _start(nk, b)
            return carry

        lax.fori_loop(0, n_full // 2, pair_body, 0)
        if n_full % 2:
            k = n_full - 1
            _wait(k, 0)
            pltpu.sync_copy(rows_v.at[0], g_hbm.at[pl.ds(base + k * GC, GC)])
        if rem:
            off = n_full * GC
            pltpu.async_copy(h_hbm.at[dst_v.at[pl.ds(off, rem)]],
                             rows_v.at[0, pl.ds(0, rem)], sem.at[0]).wait()
            pltpu.sync_copy(rows_v.at[0, pl.ds(0, rem)],
                            g_hbm.at[pl.ds(base + off, rem)])

    return sc_edges


def _agg_body(base_sref, g_ref, sc_ref, src_ref, acc_ref, sums_ref, *, nb):
    b = pl.program_id(0)

    @pl.when(b == 0)
    def _init():
        acc_ref[...] = jnp.zeros_like(acc_ref)
        sums_ref[...] = jnp.zeros_like(sums_ref)

    base = pl.multiple_of((base_sref[b] // 8) * 8, 8)
    loc = src_ref[0] - base
    iot = lax.broadcasted_iota(jnp.int32, (R_SPAN, K_EDGE), 0)
    S = jnp.where(iot == loc, sc_ref[0], 0.0).astype(jnp.bfloat16)
    contrib = jnp.dot(S, g_ref[...], preferred_element_type=jnp.float32)
    ones = jnp.ones((K_EDGE, 1), jnp.bfloat16)
    rsum = jnp.dot(S, ones, preferred_element_type=jnp.float32)
    acc_ref[pl.ds(base, R_SPAN), :] += contrib
    sums_ref[pl.ds(base, R_SPAN), :] += rsum

    @pl.when(b == nb - 1)
    def _fin():
        sv = sums_ref[...]
        acc_ref[...] = acc_ref[...] / jnp.where(sv > 0.0, sv, 1.0)


def _aggregate(base_arr, G, score3, src3, N, U):
    nb = score3.shape[0]
    NA = N + R_SPAN
    grid_spec = pltpu.PrefetchScalarGridSpec(
        num_scalar_prefetch=1,
        grid=(nb,),
        in_specs=[
            pl.BlockSpec((K_EDGE, U), lambda b, s: (b, 0)),
            pl.BlockSpec((1, 1, K_EDGE), lambda b, s: (b, 0, 0)),
            pl.BlockSpec((1, 1, K_EDGE), lambda b, s: (b, 0, 0)),
        ],
        out_specs=[
            pl.BlockSpec((NA, U), lambda b, s: (0, 0)),
            pl.BlockSpec((NA, 1), lambda b, s: (0, 0)),
        ],
    )
    acc, _ = pl.pallas_call(
        functools.partial(_agg_body, nb=nb),
        grid_spec=grid_spec,
        out_shape=[
            jax.ShapeDtypeStruct((NA, U), jnp.float32),
            jax.ShapeDtypeStruct((NA, 1), jnp.float32),
        ],
    )(base_arr, G, score3, src3)
    return acc[:N]


def kernel(inputs, edges, W, a):
    B, N, F = inputs.shape
    U = W.shape[1]
    E = edges.shape[0]
    nb = -(-E // K_EDGE)
    E_pad = nb * K_EDGE

    x = inputs.reshape(N, F)
    A2 = jnp.concatenate([a[:U], a[U:]], axis=1)

    h, s = _transform(x, W, A2, row_blk=1000)
    sT = s.T

    src = edges[:, 0]
    dst = edges[:, 1]
    pad = E_pad - E
    src_p = jnp.concatenate([src, jnp.full((pad,), N - 1, jnp.int32)])
    dst_p = jnp.concatenate([dst, jnp.zeros((pad,), jnp.int32)])

    h32 = lax.bitcast_convert_type(h.reshape(N, U // 2, 2), jnp.int32)
    sc_edges = _make_sc_edges(N, U // 2, E, E_pad)
    score, G32 = sc_edges(sT, src_p, dst_p, h32)
    G = lax.bitcast_convert_type(G32, jnp.bfloat16).reshape(E_pad, U)

    base_arr = src_p[0::K_EDGE]
    score3 = score.reshape(nb, 1, K_EDGE)
    src3 = src_p.reshape(nb, 1, K_EDGE)

    out = _aggregate(base_arr, G, score3, src3, N, U)
    return out.reshape(B, N, U)


# speedup vs baseline: 3.3936x; 3.3936x over previous
import functools

import jax
import jax.numpy as jnp
from jax import lax
from jax.experimental import pallas as pl
from jax.experimental.pallas import tpu as pltpu
from jax.experimental.pallas import tpu_sc as plsc

K_EDGE = 1024
R_SPAN = 256
GC = 128


def _rne_hi16(v):
    b = lax.bitcast_convert_type(v, jnp.int32)
    r = b + jnp.int32(0x7FFF) + ((b >> 16) & 1)
    return r


def _mm_body(x_ref, w_ref, a2_ref, h_ref, s_ref):
    h = jnp.dot(x_ref[...], w_ref[...], preferred_element_type=jnp.float32)
    s_ref[...] = jnp.dot(h, a2_ref[...], preferred_element_type=jnp.float32)
    uh = h.shape[1] // 2
    ra = _rne_hi16(h[:, :uh])
    rb = _rne_hi16(h[:, uh:])
    h_ref[...] = lax.shift_right_logical(ra, 16) | (rb & jnp.int32(-65536))


def _transform(x, W, A2, row_blk):
    N, F = x.shape
    U = W.shape[1]
    grid = (N // row_blk,)
    return pl.pallas_call(
        _mm_body,
        grid=grid,
        in_specs=[
            pl.BlockSpec((row_blk, F), lambda b: (b, 0)),
            pl.BlockSpec((F, U), lambda b: (0, 0)),
            pl.BlockSpec((U, 2), lambda b: (0, 0)),
        ],
        out_specs=[
            pl.BlockSpec((row_blk, U // 2), lambda b: (b, 0)),
            pl.BlockSpec((row_blk, 2), lambda b: (b, 0)),
        ],
        out_shape=[
            jax.ShapeDtypeStruct((N, U // 2), jnp.int32),
            jax.ShapeDtypeStruct((N, 2), jnp.float32),
        ],
    )(x, W, A2)


def _make_sc_edges(N, UW, E, E_pad):
    info = plsc.get_sparse_core_info()
    NC, NS, L = info.num_cores, info.num_subcores, info.num_lanes
    NW = NC * NS
    EC = E_pad // NW
    n_full = EC // GC
    rem = EC % GC

    mesh = plsc.VectorSubcoreMesh(core_axis_name="c", subcore_axis_name="s")

    @functools.partial(
        pl.kernel,
        mesh=mesh,
        compiler_params=pltpu.CompilerParams(needs_layout_passes=False),
        out_type=[
            jax.ShapeDtypeStruct((E_pad,), jnp.float32),
            jax.ShapeDtypeStruct((E_pad, UW), jnp.int32),
        ],
        scratch_types=[
            pltpu.VMEM((N,), jnp.float32),
            pltpu.VMEM((N,), jnp.float32),
            pltpu.VMEM((EC,), jnp.int32),
            pltpu.VMEM((EC,), jnp.int32),
            pltpu.VMEM((EC,), jnp.float32),
            pltpu.VMEM((2, GC, UW), jnp.int32),
            pltpu.SemaphoreType.DMA((2,)),
        ],
    )
    def sc_edges(s_hbm, src_hbm, dst_hbm, h_hbm, score_hbm, g_hbm,
                 s1_v, s2_v, src_v, dst_v, score_v, rows_v, sem):
        wid = lax.axis_index("s") * NC + lax.axis_index("c")
        base = wid * EC
        pltpu.sync_copy(s_hbm.at[0], s1_v)
        pltpu.sync_copy(s_hbm.at[1], s2_v)
        pltpu.sync_copy(src_hbm.at[pl.ds(base, EC)], src_v)
        pltpu.sync_copy(dst_hbm.at[pl.ds(base, EC)], dst_v)

        def score_body(i, carry):
            o = i * L
            sv = src_v[pl.ds(o, L)]
            dv = dst_v[pl.ds(o, L)]
            t = plsc.load_gather(s1_v, [sv]) + plsc.load_gather(s2_v, [dv])
            t = jnp.maximum(t, 0.2 * t)
            t = jnp.clip(t, -2.0, 2.0)
            sc = jnp.exp(t)
            gid = base + o + lax.iota(jnp.int32, L)
            sc = jnp.where(gid < E, sc, 0.0)
            score_v[pl.ds(o, L)] = sc
            return carry

        lax.fori_loop(0, EC // L, score_body, 0)
        pltpu.sync_copy(score_v, score_hbm.at[pl.ds(base, EC)])

        def _start(k, b):
            pltpu.async_copy(h_hbm.at[dst_v.at[pl.ds(k * GC, GC)]],
                             rows_v.at[b], sem.at[b])

        def _wait(k, b):
            pltpu.make_async_copy(h_hbm.at[dst_v.at[pl.ds(k * GC, GC)]],
                                  rows_v.at[b], sem.at[b]).wait()

        _start(0, 0)
        if n_full > 1:
            _start(1, 1)

        def pair_body(p, carry):
            k0 = p * 2
            for b in range(2):
                k = k0 + b
                _wait(k, b)
                pltpu.sync_copy(rows_v.at[b],
                                g_hbm.at[pl.ds(base + k * GC, GC)])
                nk = k + 2

                @pl.when(nk < n_full)
                def _():
                    _start(nk, b)
            return carry

        lax.fori_loop(0, n_full // 2, pair_body, 0)
        if n_full % 2:
            k = n_full - 1
            _wait(k, 0)
            pltpu.sync_copy(rows_v.at[0], g_hbm.at[pl.ds(base + k * GC, GC)])
        if rem:
            off = n_full * GC
            pltpu.async_copy(h_hbm.at[dst_v.at[pl.ds(off, rem)]],
                             rows_v.at[0, pl.ds(0, rem)], sem.at[0]).wait()
            pltpu.sync_copy(rows_v.at[0, pl.ds(0, rem)],
                            g_hbm.at[pl.ds(base + off, rem)])

    return sc_edges


def _agg_body(base_sref, g_ref, sc_ref, src_ref, acc_ref, sums_ref, *, nb):
    b = pl.program_id(0)

    @pl.when(b == 0)
    def _init():
        acc_ref[...] = jnp.zeros_like(acc_ref)
        sums_ref[...] = jnp.zeros_like(sums_ref)

    base = pl.multiple_of((base_sref[b] // 8) * 8, 8)
    loc = src_ref[0] - base
    iot = lax.broadcasted_iota(jnp.int32, (R_SPAN, K_EDGE), 0)
    S = jnp.where(iot == loc, sc_ref[0], 0.0)
    g32 = g_ref[...]
    uh = g32.shape[1]
    glo = lax.bitcast_convert_type(g32 << 16, jnp.float32)
    ghi = lax.bitcast_convert_type(g32 & jnp.int32(-65536), jnp.float32)
    clo = jnp.dot(S, glo, preferred_element_type=jnp.float32)
    chi = jnp.dot(S, ghi, preferred_element_type=jnp.float32)
    ones = jnp.ones((K_EDGE, 1), jnp.float32)
    rsum = jnp.dot(S, ones, preferred_element_type=jnp.float32)
    acc_ref[pl.ds(base, R_SPAN), :uh] += clo
    acc_ref[pl.ds(base, R_SPAN), uh:] += chi
    sums_ref[pl.ds(base, R_SPAN), :] += rsum

    @pl.when(b == nb - 1)
    def _fin():
        sv = sums_ref[...]
        acc_ref[...] = acc_ref[...] / jnp.where(sv > 0.0, sv, 1.0)


def _aggregate(base_arr, G, score3, src3, N, U):
    nb = score3.shape[0]
    NA = N + R_SPAN
    grid_spec = pltpu.PrefetchScalarGridSpec(
        num_scalar_prefetch=1,
        grid=(nb,),
        in_specs=[
            pl.BlockSpec((K_EDGE, U // 2), lambda b, s: (b, 0)),
            pl.BlockSpec((1, 1, K_EDGE), lambda b, s: (b, 0, 0)),
            pl.BlockSpec((1, 1, K_EDGE), lambda b, s: (b, 0, 0)),
        ],
        out_specs=[
            pl.BlockSpec((NA, U), lambda b, s: (0, 0)),
            pl.BlockSpec((NA, 1), lambda b, s: (0, 0)),
        ],
    )
    acc, _ = pl.pallas_call(
        functools.partial(_agg_body, nb=nb),
        grid_spec=grid_spec,
        out_shape=[
            jax.ShapeDtypeStruct((NA, U), jnp.float32),
            jax.ShapeDtypeStruct((NA, 1), jnp.float32),
        ],
    )(base_arr, G, score3, src3)
    return acc[:N]


def kernel(inputs, edges, W, a):
    B, N, F = inputs.shape
    U = W.shape[1]
    E = edges.shape[0]
    nb = -(-E // K_EDGE)
    E_pad = nb * K_EDGE

    x = inputs.reshape(N, F)
    A2 = jnp.concatenate([a[:U], a[U:]], axis=1)

    h, s = _transform(x, W, A2, row_blk=1000)
    sT = s.T

    src = edges[:, 0]
    dst = edges[:, 1]
    pad = E_pad - E
    src_p = jnp.concatenate([src, jnp.full((pad,), N - 1, jnp.int32)])
    dst_p = jnp.concatenate([dst, jnp.zeros((pad,), jnp.int32)])

    sc_edges = _make_sc_edges(N, U // 2, E, E_pad)
    score, G = sc_edges(sT, src_p, dst_p, h)

    base_arr = src_p[0::K_EDGE]
    score3 = score.reshape(nb, 1, K_EDGE)
    src3 = src_p.reshape(nb, 1, K_EDGE)

    out = _aggregate(base_arr, G, score3, src3, N, U)
    return out.reshape(B, N, U)


# speedup vs baseline: 3.4013x; 1.0022x over previous
import functools

import jax
import jax.numpy as jnp
from jax import lax
from jax.experimental import pallas as pl
from jax.experimental.pallas import tpu as pltpu
from jax.experimental.pallas import tpu_sc as plsc

K_EDGE = 1024
R_SPAN = 256
GC = 128


def _rne_hi16(v):
    b = lax.bitcast_convert_type(v, jnp.int32)
    r = b + jnp.int32(0x7FFF) + ((b >> 16) & 1)
    return r


def _mm_body(x_ref, w_ref, a2_ref, h_ref, s_ref):
    h = jnp.dot(x_ref[...], w_ref[...], preferred_element_type=jnp.float32)
    s_ref[...] = jnp.dot(h, a2_ref[...], preferred_element_type=jnp.float32)
    uh = h.shape[1] // 2
    ra = _rne_hi16(h[:, :uh])
    rb = _rne_hi16(h[:, uh:])
    h_ref[...] = lax.shift_right_logical(ra, 16) | (rb & jnp.int32(-65536))


def _transform(x, W, A2, row_blk):
    N, F = x.shape
    U = W.shape[1]
    grid = (N // row_blk,)
    return pl.pallas_call(
        _mm_body,
        grid=grid,
        in_specs=[
            pl.BlockSpec((row_blk, F), lambda b: (b, 0)),
            pl.BlockSpec((F, U), lambda b: (0, 0)),
            pl.BlockSpec((U, 2), lambda b: (0, 0)),
        ],
        out_specs=[
            pl.BlockSpec((row_blk, U // 2), lambda b: (b, 0)),
            pl.BlockSpec((row_blk, 2), lambda b: (b, 0)),
        ],
        out_shape=[
            jax.ShapeDtypeStruct((N, U // 2), jnp.int32),
            jax.ShapeDtypeStruct((N, 2), jnp.float32),
        ],
    )(x, W, A2)


def _make_sc_edges(N, UW, n_real, E_pad):
    info = plsc.get_sparse_core_info()
    NC, NS, L = info.num_cores, info.num_subcores, info.num_lanes
    NW = NC * NS
    EC = E_pad // NW
    n_full = EC // GC
    rem = EC % GC

    mesh = plsc.VectorSubcoreMesh(core_axis_name="c", subcore_axis_name="s")

    @functools.partial(
        pl.kernel,
        mesh=mesh,
        compiler_params=pltpu.CompilerParams(needs_layout_passes=False),
        out_type=[
            jax.ShapeDtypeStruct((E_pad,), jnp.float32),
            jax.ShapeDtypeStruct((E_pad, UW), jnp.int32),
        ],
        scratch_types=[
            pltpu.VMEM((N,), jnp.float32),
            pltpu.VMEM((N,), jnp.float32),
            pltpu.VMEM((EC,), jnp.int32),
            pltpu.VMEM((EC,), jnp.int32),
            pltpu.VMEM((EC,), jnp.float32),
            pltpu.VMEM((2, GC, UW), jnp.int32),
            pltpu.SemaphoreType.DMA((2,)),
        ],
    )
    def sc_edges(s_hbm, src_hbm, dst_hbm, h_hbm, score_hbm, g_hbm,
                 s1_v, s2_v, src_v, dst_v, score_v, rows_v, sem):
        wid = lax.axis_index("s") * NC + lax.axis_index("c")
        base = wid * EC
        pltpu.sync_copy(s_hbm.at[0], s1_v)
        pltpu.sync_copy(s_hbm.at[1], s2_v)
        pltpu.sync_copy(src_hbm.at[pl.ds(base, EC)], src_v)
        pltpu.sync_copy(dst_hbm.at[pl.ds(base, EC)], dst_v)

        def score_body(i, carry):
            o = i * L
            sv = src_v[pl.ds(o, L)]
            dv = dst_v[pl.ds(o, L)]
            t = plsc.load_gather(s1_v, [sv]) + plsc.load_gather(s2_v, [dv])
            t = jnp.maximum(t, 0.2 * t)
            t = jnp.clip(t, -2.0, 2.0)
            sc = jnp.exp(t)
            gid = base + o + lax.iota(jnp.int32, L)
            sc = jnp.where(gid < n_real, sc, 0.0)
            score_v[pl.ds(o, L)] = sc
            return carry

        lax.fori_loop(0, EC // L, score_body, 0)
        pltpu.sync_copy(score_v, score_hbm.at[pl.ds(base, EC)])

        def _start(k, b):
            pltpu.async_copy(h_hbm.at[dst_v.at[pl.ds(k * GC, GC)]],
                             rows_v.at[b], sem.at[b])

        def _wait(k, b):
            pltpu.make_async_copy(h_hbm.at[dst_v.at[pl.ds(k * GC, GC)]],
                                  rows_v.at[b], sem.at[b]).wait()

        _start(0, 0)
        if n_full > 1:
            _start(1, 1)

        def pair_body(p, carry):
            k0 = p * 2
            for b in range(2):
                k = k0 + b
                _wait(k, b)
                pltpu.sync_copy(rows_v.at[b],
                                g_hbm.at[pl.ds(base + k * GC, GC)])
                nk = k + 2

                @pl.when(nk < n_full)
                def _():
                    _start(nk, b)
            return carry

        lax.fori_loop(0, n_full // 2, pair_body, 0)
        if n_full % 2:
            k = n_full - 1
            _wait(k, 0)
            pltpu.sync_copy(rows_v.at[0], g_hbm.at[pl.ds(base + k * GC, GC)])
        if rem:
            off = n_full * GC
            pltpu.async_copy(h_hbm.at[dst_v.at[pl.ds(off, rem)]],
                             rows_v.at[0, pl.ds(0, rem)], sem.at[0]).wait()
            pltpu.sync_copy(rows_v.at[0, pl.ds(0, rem)],
                            g_hbm.at[pl.ds(base + off, rem)])

    return sc_edges


def _agg_body(base_sref, g_ref, sc_ref, src_ref, acc_ref, sums_ref):
    b = pl.program_id(0)

    @pl.when(b == 0)
    def _init():
        acc_ref[...] = jnp.zeros_like(acc_ref)
        sums_ref[...] = jnp.zeros_like(sums_ref)

    base = pl.multiple_of((base_sref[b] // 8) * 8, 8)
    loc = src_ref[0] - base
    iot = lax.broadcasted_iota(jnp.int32, (R_SPAN, K_EDGE), 0)
    S = jnp.where(iot == loc, sc_ref[0], 0.0)
    g32 = g_ref[...]
    uh = g32.shape[1]
    glo = lax.bitcast_convert_type(g32 << 16, jnp.float32)
    ghi = lax.bitcast_convert_type(g32 & jnp.int32(-65536), jnp.float32)
    clo = jnp.dot(S, glo, preferred_element_type=jnp.float32)
    chi = jnp.dot(S, ghi, preferred_element_type=jnp.float32)
    ones = jnp.ones((K_EDGE, 1), jnp.float32)
    rsum = jnp.dot(S, ones, preferred_element_type=jnp.float32)
    acc_ref[pl.ds(base, R_SPAN), :uh] += clo
    acc_ref[pl.ds(base, R_SPAN), uh:] += chi
    sums_ref[pl.ds(base, R_SPAN), :] += rsum


def _aggregate(base_arr, G, score3, src3, N, U):
    nb = score3.shape[0]
    NA = N + R_SPAN
    grid_spec = pltpu.PrefetchScalarGridSpec(
        num_scalar_prefetch=1,
        grid=(nb,),
        in_specs=[
            pl.BlockSpec((K_EDGE, U // 2), lambda b, s: (b, 0)),
            pl.BlockSpec((1, 1, K_EDGE), lambda b, s: (b, 0, 0)),
            pl.BlockSpec((1, 1, K_EDGE), lambda b, s: (b, 0, 0)),
        ],
        out_specs=[
            pl.BlockSpec((NA, U), lambda b, s: (0, 0)),
            pl.BlockSpec((NA, 1), lambda b, s: (0, 0)),
        ],
    )
    return pl.pallas_call(
        _agg_body,
        grid_spec=grid_spec,
        out_shape=[
            jax.ShapeDtypeStruct((NA, U), jnp.float32),
            jax.ShapeDtypeStruct((NA, 1), jnp.float32),
        ],
    )(base_arr, G, score3, src3)


def _combine_body(a1_ref, a2_ref, s1_ref, s2_ref, out_ref):
    sv = s1_ref[...] + s2_ref[...]
    out_ref[...] = (a1_ref[...] + a2_ref[...]) / jnp.where(sv > 0.0, sv, 1.0)


def _combine(acc1, sums1, acc2, sums2, N, U, row_blk):
    grid = (N // row_blk,)
    return pl.pallas_call(
        _combine_body,
        grid=grid,
        in_specs=[
            pl.BlockSpec((row_blk, U), lambda b: (b, 0)),
            pl.BlockSpec((row_blk, U), lambda b: (b, 0)),
            pl.BlockSpec((row_blk, 1), lambda b: (b, 0)),
            pl.BlockSpec((row_blk, 1), lambda b: (b, 0)),
        ],
        out_specs=pl.BlockSpec((row_blk, U), lambda b: (b, 0)),
        out_shape=jax.ShapeDtypeStruct((N, U), jnp.float32),
    )(acc1, acc2, sums1, sums2)


def kernel(inputs, edges, W, a):
    B, N, F = inputs.shape
    U = W.shape[1]
    E = edges.shape[0]
    nb = -(-E // K_EDGE)
    E_pad = nb * K_EDGE

    x = inputs.reshape(N, F)
    A2 = jnp.concatenate([a[:U], a[U:]], axis=1)

    h, s = _transform(x, W, A2, row_blk=1000)
    sT = s.T

    src = edges[:, 0]
    dst = edges[:, 1]
    pad = E_pad - E
    src_p = jnp.concatenate([src, jnp.full((pad,), N - 1, jnp.int32)])
    dst_p = jnp.concatenate([dst, jnp.zeros((pad,), jnp.int32)])

    nb1 = nb // 2
    splits = [(0, nb1), (nb1, nb)]
    parts = []
    for (b0, b1) in splits:
        e0, e1 = b0 * K_EDGE, b1 * K_EDGE
        n_real = max(0, min(E, e1) - e0)
        sc_edges = _make_sc_edges(N, U // 2, n_real, e1 - e0)
        parts.append((e0, e1, sc_edges))

    scored = [(e0, e1, sce(sT, src_p[e0:e1], dst_p[e0:e1], h))
              for (e0, e1, sce) in parts]

    accs = []
    for (e0, e1, (score, G)) in scored:
        nbi = (e1 - e0) // K_EDGE
        base_arr = src_p[e0:e1:K_EDGE]
        score3 = score.reshape(nbi, 1, K_EDGE)
        src3 = src_p[e0:e1].reshape(nbi, 1, K_EDGE)
        accs.append(_aggregate(base_arr, G, score3, src3, N, U))

    (acc1, sums1), (acc2, sums2) = accs
    out = _combine(acc1, sums1, acc2, sums2, N, U, row_blk=1000)
    return out.reshape(B, N, U)


# speedup vs baseline: 3.5685x; 1.0492x over previous
import functools

import jax
import jax.numpy as jnp
from jax import lax
from jax.experimental import pallas as pl
from jax.experimental.pallas import tpu as pltpu
from jax.experimental.pallas import tpu_sc as plsc

K_EDGE = 1024
R_SPAN = 128
GC = 128


def _rne_hi16(v):
    b = lax.bitcast_convert_type(v, jnp.int32)
    r = b + jnp.int32(0x7FFF) + ((b >> 16) & 1)
    return r


def _mm_body(x_ref, w_ref, a2_ref, h_ref, s_ref):
    h = jnp.dot(x_ref[...], w_ref[...], preferred_element_type=jnp.float32)
    s_ref[...] = jnp.dot(h, a2_ref[...], preferred_element_type=jnp.float32)
    uh = h.shape[1] // 2
    ra = _rne_hi16(h[:, :uh])
    rb = _rne_hi16(h[:, uh:])
    h_ref[...] = lax.shift_right_logical(ra, 16) | (rb & jnp.int32(-65536))


def _transform(x, W, A2, row_blk):
    N, F = x.shape
    U = W.shape[1]
    grid = (N // row_blk,)
    return pl.pallas_call(
        _mm_body,
        grid=grid,
        in_specs=[
            pl.BlockSpec((row_blk, F), lambda b: (b, 0)),
            pl.BlockSpec((F, U), lambda b: (0, 0)),
            pl.BlockSpec((U, 2), lambda b: (0, 0)),
        ],
        out_specs=[
            pl.BlockSpec((row_blk, U // 2), lambda b: (b, 0)),
            pl.BlockSpec((row_blk, 2), lambda b: (b, 0)),
        ],
        out_shape=[
            jax.ShapeDtypeStruct((N, U // 2), jnp.int32),
            jax.ShapeDtypeStruct((N, 2), jnp.float32),
        ],
    )(x, W, A2)


def _make_sc_edges(N, UW, n_real, E_pad):
    info = plsc.get_sparse_core_info()
    NC, NS, L = info.num_cores, info.num_subcores, info.num_lanes
    NW = NC * NS
    EC = E_pad // NW
    n_full = EC // GC
    rem = EC % GC

    mesh = plsc.VectorSubcoreMesh(core_axis_name="c", subcore_axis_name="s")

    @functools.partial(
        pl.kernel,
        mesh=mesh,
        compiler_params=pltpu.CompilerParams(needs_layout_passes=False),
        out_type=[
            jax.ShapeDtypeStruct((E_pad,), jnp.float32),
            jax.ShapeDtypeStruct((E_pad, UW), jnp.int32),
        ],
        scratch_types=[
            pltpu.VMEM((N,), jnp.float32),
            pltpu.VMEM((N,), jnp.float32),
            pltpu.VMEM((EC,), jnp.int32),
            pltpu.VMEM((EC,), jnp.int32),
            pltpu.VMEM((EC,), jnp.float32),
            pltpu.VMEM((2, GC, UW), jnp.int32),
            pltpu.SemaphoreType.DMA((2,)),
        ],
    )
    def sc_edges(s_hbm, src_hbm, dst_hbm, h_hbm, score_hbm, g_hbm,
                 s1_v, s2_v, src_v, dst_v, score_v, rows_v, sem):
        wid = lax.axis_index("s") * NC + lax.axis_index("c")
        base = wid * EC
        pltpu.sync_copy(s_hbm.at[0], s1_v)
        pltpu.sync_copy(s_hbm.at[1], s2_v)
        pltpu.sync_copy(src_hbm.at[pl.ds(base, EC)], src_v)
        pltpu.sync_copy(dst_hbm.at[pl.ds(base, EC)], dst_v)

        def score_body(i, carry):
            o = i * L
            sv = src_v[pl.ds(o, L)]
            dv = dst_v[pl.ds(o, L)]
            t = plsc.load_gather(s1_v, [sv]) + plsc.load_gather(s2_v, [dv])
            t = jnp.maximum(t, 0.2 * t)
            t = jnp.clip(t, -2.0, 2.0)
            sc = jnp.exp(t)
            gid = base + o + lax.iota(jnp.int32, L)
            sc = jnp.where(gid < n_real, sc, 0.0)
            score_v[pl.ds(o, L)] = sc
            return carry

        lax.fori_loop(0, EC // L, score_body, 0)
        pltpu.sync_copy(score_v, score_hbm.at[pl.ds(base, EC)])

        def _start(k, b):
            pltpu.async_copy(h_hbm.at[dst_v.at[pl.ds(k * GC, GC)]],
                             rows_v.at[b], sem.at[b])

        def _wait(k, b):
            pltpu.make_async_copy(h_hbm.at[dst_v.at[pl.ds(k * GC, GC)]],
                                  rows_v.at[b], sem.at[b]).wait()

        _start(0, 0)
        if n_full > 1:
            _start(1, 1)

        def pair_body(p, carry):
            k0 = p * 2
            for b in range(2):
                k = k0 + b
                _wait(k, b)
                pltpu.sync_copy(rows_v.at[b],
                                g_hbm.at[pl.ds(base + k * GC, GC)])
                nk = k + 2

                @pl.when(nk < n_full)
                def _():
                    _start(nk, b)
            return carry

        lax.fori_loop(0, n_full // 2, pair_body, 0)
        if n_full % 2:
            k = n_full - 1
            _wait(k, 0)
            pltpu.sync_copy(rows_v.at[0], g_hbm.at[pl.ds(base + k * GC, GC)])
        if rem:
            off = n_full * GC
            pltpu.async_copy(h_hbm.at[dst_v.at[pl.ds(off, rem)]],
                             rows_v.at[0, pl.ds(0, rem)], sem.at[0]).wait()
            pltpu.sync_copy(rows_v.at[0, pl.ds(0, rem)],
                            g_hbm.at[pl.ds(base + off, rem)])

    return sc_edges


def _agg_body(base_sref, g_ref, sc_ref, src_ref, acc_ref, sums_ref):
    b = pl.program_id(0)

    @pl.when(b == 0)
    def _init():
        acc_ref[...] = jnp.zeros_like(acc_ref)
        sums_ref[...] = jnp.zeros_like(sums_ref)

    base = pl.multiple_of((base_sref[b] // 8) * 8, 8)
    loc = src_ref[0] - base
    iot = lax.broadcasted_iota(jnp.int32, (R_SPAN, K_EDGE), 0)
    S = jnp.where(iot == loc, sc_ref[0], 0.0).astype(jnp.bfloat16)
    g32 = g_ref[...]
    uh = g32.shape[1]
    glo = lax.bitcast_convert_type(g32 << 16, jnp.float32).astype(jnp.bfloat16)
    ghi = lax.bitcast_convert_type(g32 & jnp.int32(-65536),
                                   jnp.float32).astype(jnp.bfloat16)
    clo = jnp.dot(S, glo, preferred_element_type=jnp.float32)
    chi = jnp.dot(S, ghi, preferred_element_type=jnp.float32)
    ones = jnp.ones((K_EDGE, 1), jnp.bfloat16)
    rsum = jnp.dot(S, ones, preferred_element_type=jnp.float32)
    acc_ref[pl.ds(base, R_SPAN), :uh] += clo
    acc_ref[pl.ds(base, R_SPAN), uh:] += chi
    sums_ref[pl.ds(base, R_SPAN), :] += rsum


def _aggregate(base_arr, G, score3, src3, N, U):
    nb = score3.shape[0]
    NA = N + R_SPAN
    grid_spec = pltpu.PrefetchScalarGridSpec(
        num_scalar_prefetch=1,
        grid=(nb,),
        in_specs=[
            pl.BlockSpec((K_EDGE, U // 2), lambda b, s: (b, 0)),
            pl.BlockSpec((1, 1, K_EDGE), lambda b, s: (b, 0, 0)),
            pl.BlockSpec((1, 1, K_EDGE), lambda b, s: (b, 0, 0)),
        ],
        out_specs=[
            pl.BlockSpec((NA, U), lambda b, s: (0, 0)),
            pl.BlockSpec((NA, 1), lambda b, s: (0, 0)),
        ],
    )
    return pl.pallas_call(
        _agg_body,
        grid_spec=grid_spec,
        out_shape=[
            jax.ShapeDtypeStruct((NA, U), jnp.float32),
            jax.ShapeDtypeStruct((NA, 1), jnp.float32),
        ],
    )(base_arr, G, score3, src3)


def _combine_body(a1_ref, a2_ref, s1_ref, s2_ref, out_ref):
    sv = s1_ref[...] + s2_ref[...]
    out_ref[...] = (a1_ref[...] + a2_ref[...]) / jnp.where(sv > 0.0, sv, 1.0)


def _combine(acc1, sums1, acc2, sums2, N, U, row_blk):
    grid = (N // row_blk,)
    return pl.pallas_call(
        _combine_body,
        grid=grid,
        in_specs=[
            pl.BlockSpec((row_blk, U), lambda b: (b, 0)),
            pl.BlockSpec((row_blk, U), lambda b: (b, 0)),
            pl.BlockSpec((row_blk, 1), lambda b: (b, 0)),
            pl.BlockSpec((row_blk, 1), lambda b: (b, 0)),
        ],
        out_specs=pl.BlockSpec((row_blk, U), lambda b: (b, 0)),
        out_shape=jax.ShapeDtypeStruct((N, U), jnp.float32),
    )(acc1, acc2, sums1, sums2)


def kernel(inputs, edges, W, a):
    B, N, F = inputs.shape
    U = W.shape[1]
    E = edges.shape[0]
    nb = -(-E // K_EDGE)
    E_pad = nb * K_EDGE

    x = inputs.reshape(N, F)
    A2 = jnp.concatenate([a[:U], a[U:]], axis=1)

    h, s = _transform(x, W, A2, row_blk=1000)
    sT = s.T

    src = edges[:, 0]
    dst = edges[:, 1]
    pad = E_pad - E
    src_p = jnp.concatenate([src, jnp.full((pad,), N - 1, jnp.int32)])
    dst_p = jnp.concatenate([dst, jnp.zeros((pad,), jnp.int32)])

    nb1 = nb // 2
    splits = [(0, nb1), (nb1, nb)]
    parts = []
    for (b0, b1) in splits:
        e0, e1 = b0 * K_EDGE, b1 * K_EDGE
        n_real = max(0, min(E, e1) - e0)
        sc_edges = _make_sc_edges(N, U // 2, n_real, e1 - e0)
        parts.append((e0, e1, sc_edges))

    scored = [(e0, e1, sce(sT, src_p[e0:e1], dst_p[e0:e1], h))
              for (e0, e1, sce) in parts]

    accs = []
    for (e0, e1, (score, G)) in scored:
        nbi = (e1 - e0) // K_EDGE
        base_arr = src_p[e0:e1:K_EDGE]
        score3 = score.reshape(nbi, 1, K_EDGE)
        src3 = src_p[e0:e1].reshape(nbi, 1, K_EDGE)
        accs.append(_aggregate(base_arr, G, score3, src3, N, U))

    (acc1, sums1), (acc2, sums2) = accs
    out = _combine(acc1, sums1, acc2, sums2, N, U, row_blk=1000)
    return out.reshape(B, N, U)


# speedup vs baseline: 3.5864x; 1.0050x over previous
import functools

import jax
import jax.numpy as jnp
from jax import lax
from jax.experimental import pallas as pl
from jax.experimental.pallas import tpu as pltpu
from jax.experimental.pallas import tpu_sc as plsc

K_EDGE = 1024
R_SPAN = 128
GC = 128


def _rne_hi16(v):
    b = lax.bitcast_convert_type(v, jnp.int32)
    r = b + jnp.int32(0x7FFF) + ((b >> 16) & 1)
    return r


def _mm_body(x_ref, w_ref, a2_ref, h_ref, s_ref):
    h = jnp.dot(x_ref[...], w_ref[...], preferred_element_type=jnp.float32)
    s_ref[...] = jnp.dot(h, a2_ref[...], preferred_element_type=jnp.float32)
    uh = h.shape[1] // 2
    ra = _rne_hi16(h[:, :uh])
    rb = _rne_hi16(h[:, uh:])
    h_ref[...] = lax.shift_right_logical(ra, 16) | (rb & jnp.int32(-65536))


def _transform(x, W, A2, row_blk):
    N, F = x.shape
    U = W.shape[1]
    grid = (N // row_blk,)
    return pl.pallas_call(
        _mm_body,
        grid=grid,
        in_specs=[
            pl.BlockSpec((row_blk, F), lambda b: (b, 0)),
            pl.BlockSpec((F, U), lambda b: (0, 0)),
            pl.BlockSpec((U, 2), lambda b: (0, 0)),
        ],
        out_specs=[
            pl.BlockSpec((row_blk, U // 2), lambda b: (b, 0)),
            pl.BlockSpec((row_blk, 2), lambda b: (b, 0)),
        ],
        out_shape=[
            jax.ShapeDtypeStruct((N, U // 2), jnp.int32),
            jax.ShapeDtypeStruct((N, 2), jnp.float32),
        ],
    )(x, W, A2)


def _make_sc_edges(N, UW, n_real, E_pad):
    info = plsc.get_sparse_core_info()
    NC, NS, L = info.num_cores, info.num_subcores, info.num_lanes
    NW = NC * NS
    EC = E_pad // NW
    n_full = EC // GC
    rem = EC % GC

    mesh = plsc.VectorSubcoreMesh(core_axis_name="c", subcore_axis_name="s")

    @functools.partial(
        pl.kernel,
        mesh=mesh,
        compiler_params=pltpu.CompilerParams(needs_layout_passes=False),
        out_type=[
            jax.ShapeDtypeStruct((E_pad,), jnp.float32),
            jax.ShapeDtypeStruct((E_pad, UW), jnp.int32),
        ],
        scratch_types=[
            pltpu.VMEM((N,), jnp.float32),
            pltpu.VMEM((N,), jnp.float32),
            pltpu.VMEM((EC,), jnp.int32),
            pltpu.VMEM((EC,), jnp.int32),
            pltpu.VMEM((EC,), jnp.float32),
            pltpu.VMEM((2, GC, UW), jnp.int32),
            pltpu.SemaphoreType.DMA((2,)),
        ],
    )
    def sc_edges(s_hbm, src_hbm, dst_hbm, h_hbm, score_hbm, g_hbm,
                 s1_v, s2_v, src_v, dst_v, score_v, rows_v, sem):
        wid = lax.axis_index("s") * NC + lax.axis_index("c")
        base = wid * EC
        pltpu.sync_copy(s_hbm.at[0], s1_v)
        pltpu.sync_copy(s_hbm.at[1], s2_v)
        pltpu.sync_copy(src_hbm.at[pl.ds(base, EC)], src_v)
        pltpu.sync_copy(dst_hbm.at[pl.ds(base, EC)], dst_v)

        def score_body(i, carry):
            o = i * L
            sv = src_v[pl.ds(o, L)]
            dv = dst_v[pl.ds(o, L)]
            t = plsc.load_gather(s1_v, [sv]) + plsc.load_gather(s2_v, [dv])
            t = jnp.maximum(t, 0.2 * t)
            t = jnp.clip(t, -2.0, 2.0)
            sc = jnp.exp(t)
            gid = base + o + lax.iota(jnp.int32, L)
            sc = jnp.where(gid < n_real, sc, 0.0)
            score_v[pl.ds(o, L)] = sc
            return carry

        lax.fori_loop(0, EC // L, score_body, 0)
        pltpu.sync_copy(score_v, score_hbm.at[pl.ds(base, EC)])

        def _start(k, b):
            pltpu.async_copy(h_hbm.at[dst_v.at[pl.ds(k * GC, GC)]],
                             rows_v.at[b], sem.at[b])

        def _wait(k, b):
            pltpu.make_async_copy(h_hbm.at[dst_v.at[pl.ds(k * GC, GC)]],
                                  rows_v.at[b], sem.at[b]).wait()

        _start(0, 0)
        if n_full > 1:
            _start(1, 1)

        def pair_body(p, carry):
            k0 = p * 2
            for b in range(2):
                k = k0 + b
                _wait(k, b)
                pltpu.sync_copy(rows_v.at[b],
                                g_hbm.at[pl.ds(base + k * GC, GC)])
                nk = k + 2

                @pl.when(nk < n_full)
                def _():
                    _start(nk, b)
            return carry

        lax.fori_loop(0, n_full // 2, pair_body, 0)
        if n_full % 2:
            k = n_full - 1
            _wait(k, 0)
            pltpu.sync_copy(rows_v.at[0], g_hbm.at[pl.ds(base + k * GC, GC)])
        if rem:
            off = n_full * GC
            pltpu.async_copy(h_hbm.at[dst_v.at[pl.ds(off, rem)]],
                             rows_v.at[0, pl.ds(0, rem)], sem.at[0]).wait()
            pltpu.sync_copy(rows_v.at[0, pl.ds(0, rem)],
                            g_hbm.at[pl.ds(base + off, rem)])

    return sc_edges


def _agg_body(base_sref, g_ref, sc_ref, src_ref, acc_ref, sums_ref):
    b = pl.program_id(0)

    @pl.when(b == 0)
    def _init():
        acc_ref[...] = jnp.zeros_like(acc_ref)
        sums_ref[...] = jnp.zeros_like(sums_ref)

    base = pl.multiple_of((base_sref[b] // 8) * 8, 8)
    loc = src_ref[0] - base
    iot = lax.broadcasted_iota(jnp.int32, (R_SPAN, K_EDGE), 0)
    S = jnp.where(iot == loc, sc_ref[0], 0.0)
    g32 = g_ref[...]
    uh = g32.shape[1]
    glo = lax.bitcast_convert_type(g32 << 16, jnp.float32)
    ghi = lax.bitcast_convert_type(g32 & jnp.int32(-65536), jnp.float32)
    clo = jnp.dot(S, glo, preferred_element_type=jnp.float32)
    chi = jnp.dot(S, ghi, preferred_element_type=jnp.float32)
    ones = jnp.ones((K_EDGE, 1), jnp.float32)
    rsum = jnp.dot(S, ones, preferred_element_type=jnp.float32)
    acc_ref[pl.ds(base, R_SPAN), :uh] += clo
    acc_ref[pl.ds(base, R_SPAN), uh:] += chi
    sums_ref[pl.ds(base, R_SPAN), :] += rsum


def _aggregate(base_arr, G, score3, src3, N, U):
    nb = score3.shape[0]
    NA = N + R_SPAN
    grid_spec = pltpu.PrefetchScalarGridSpec(
        num_scalar_prefetch=1,
        grid=(nb,),
        in_specs=[
            pl.BlockSpec((K_EDGE, U // 2), lambda b, s: (b, 0)),
            pl.BlockSpec((1, 1, K_EDGE), lambda b, s: (b, 0, 0)),
            pl.BlockSpec((1, 1, K_EDGE), lambda b, s: (b, 0, 0)),
        ],
        out_specs=[
            pl.BlockSpec((NA, U), lambda b, s: (0, 0)),
            pl.BlockSpec((NA, 1), lambda b, s: (0, 0)),
        ],
    )
    return pl.pallas_call(
        _agg_body,
        grid_spec=grid_spec,
        out_shape=[
            jax.ShapeDtypeStruct((NA, U), jnp.float32),
            jax.ShapeDtypeStruct((NA, 1), jnp.float32),
        ],
    )(base_arr, G, score3, src3)


def _combine_body(a1_ref, a2_ref, s1_ref, s2_ref, out_ref):
    sv = s1_ref[...] + s2_ref[...]
    out_ref[...] = (a1_ref[...] + a2_ref[...]) / jnp.where(sv > 0.0, sv, 1.0)


def _combine(acc1, sums1, acc2, sums2, N, U, row_blk):
    grid = (N // row_blk,)
    return pl.pallas_call(
        _combine_body,
        grid=grid,
        in_specs=[
            pl.BlockSpec((row_blk, U), lambda b: (b, 0)),
            pl.BlockSpec((row_blk, U), lambda b: (b, 0)),
            pl.BlockSpec((row_blk, 1), lambda b: (b, 0)),
            pl.BlockSpec((row_blk, 1), lambda b: (b, 0)),
        ],
        out_specs=pl.BlockSpec((row_blk, U), lambda b: (b, 0)),
        out_shape=jax.ShapeDtypeStruct((N, U), jnp.float32),
    )(acc1, acc2, sums1, sums2)


def kernel(inputs, edges, W, a):
    B, N, F = inputs.shape
    U = W.shape[1]
    E = edges.shape[0]
    nb = -(-E // K_EDGE)
    E_pad = nb * K_EDGE

    x = inputs.reshape(N, F)
    A2 = jnp.concatenate([a[:U], a[U:]], axis=1)

    h, s = _transform(x, W, A2, row_blk=1000)
    sT = s.T

    src = edges[:, 0]
    dst = edges[:, 1]
    pad = E_pad - E
    src_p = jnp.concatenate([src, jnp.full((pad,), N - 1, jnp.int32)])
    dst_p = jnp.concatenate([dst, jnp.zeros((pad,), jnp.int32)])

    nb1 = nb // 2
    splits = [(0, nb1), (nb1, nb)]
    parts = []
    for (b0, b1) in splits:
        e0, e1 = b0 * K_EDGE, b1 * K_EDGE
        n_real = max(0, min(E, e1) - e0)
        sc_edges = _make_sc_edges(N, U // 2, n_real, e1 - e0)
        parts.append((e0, e1, sc_edges))

    scored = [(e0, e1, sce(sT, src_p[e0:e1], dst_p[e0:e1], h))
              for (e0, e1, sce) in parts]

    accs = []
    for (e0, e1, (score, G)) in scored:
        nbi = (e1 - e0) // K_EDGE
        base_arr = src_p[e0:e1:K_EDGE]
        score3 = score.reshape(nbi, 1, K_EDGE)
        src3 = src_p[e0:e1].reshape(nbi, 1, K_EDGE)
        accs.append(_aggregate(base_arr, G, score3, src3, N, U))

    (acc1, sums1), (acc2, sums2) = accs
    out = _combine(acc1, sums1, acc2, sums2, N, U, row_blk=1000)
    return out.reshape(B, N, U)
